# baseline (device time: 43735 ns/iter reference)
import jax
import jax.numpy as jnp
from jax import lax
from jax.experimental import pallas as pl
from jax.experimental.pallas import tpu as pltpu

N_DEV = 4
N_HOPS = N_DEV - 1
N_SEG = 4


def kernel(x):
    m_per, n = x.shape
    m_half = m_per // 2
    m_seg = m_half // N_SEG

    def body(x_ref, out_ref, cw_ref, ccw_ref,
             send_cw, recv_cw, send_ccw, recv_ccw):
        my_pos = lax.axis_index("i")
        left = (my_pos - 1) % N_DEV
        right = (my_pos + 1) % N_DEV

        barrier_sem = pltpu.get_barrier_semaphore()
        for nbr in [left, right]:
            pl.semaphore_signal(
                barrier_sem, inc=1,
                device_id=(nbr,), device_id_type=pl.DeviceIdType.MESH,
            )
        pl.semaphore_wait(barrier_sem, 2)

        def make_pair(h, s):
            if h == 0:
                src_cw = x_ref.at[pl.ds(s * m_seg, m_seg), :]
                src_ccw = x_ref.at[pl.ds(m_half + s * m_seg, m_seg), :]
            else:
                src_cw = cw_ref.at[h - 1, pl.ds(s * m_seg, m_seg), :]
                src_ccw = ccw_ref.at[h - 1, pl.ds(s * m_seg, m_seg), :]
            rdma_cw = pltpu.make_async_remote_copy(
                src_ref=src_cw,
                dst_ref=cw_ref.at[h, pl.ds(s * m_seg, m_seg), :],
                send_sem=send_cw.at[h, s],
                recv_sem=recv_cw.at[h, s],
                device_id=(right,),
                device_id_type=pl.DeviceIdType.MESH,
            )
            rdma_ccw = pltpu.make_async_remote_copy(
                src_ref=src_ccw,
                dst_ref=ccw_ref.at[h, pl.ds(s * m_seg, m_seg), :],
                send_sem=send_ccw.at[h, s],
                recv_sem=recv_ccw.at[h, s],
                device_id=(left,),
                device_id_type=pl.DeviceIdType.MESH,
            )
            return rdma_cw, rdma_ccw

        def store_hop(h):
            o_cw = (my_pos - h - 1) % N_DEV
            o_ccw = (my_pos + h + 1) % N_DEV
            out_ref[pl.ds(o_cw * m_per, m_half), :] = cw_ref[h]
            out_ref[pl.ds(o_ccw * m_per + m_half, m_half), :] = ccw_ref[h]

        rdmas = {}
        for s in range(N_SEG):
            pair = make_pair(0, s)
            pair[0].start()
            pair[1].start()
            rdmas[(0, s)] = pair

        out_ref[pl.ds(my_pos * m_per, m_per), :] = x_ref[:, :]

        for h in range(1, N_HOPS):
            for s in range(N_SEG):
                rdmas[(h - 1, s)][0].wait_recv()
                rdmas[(h - 1, s)][1].wait_recv()
                pair = make_pair(h, s)
                pair[0].start()
                pair[1].start()
                rdmas[(h, s)] = pair
            store_hop(h - 1)

        o_cw = (my_pos - N_HOPS) % N_DEV
        o_ccw = (my_pos + N_HOPS) % N_DEV
        for s in range(N_SEG):
            rdmas[(N_HOPS - 1, s)][0].wait_recv()
            rdmas[(N_HOPS - 1, s)][1].wait_recv()
            out_ref[pl.ds(o_cw * m_per + s * m_seg, m_seg), :] = (
                cw_ref[N_HOPS - 1, pl.ds(s * m_seg, m_seg), :])
            out_ref[pl.ds(o_ccw * m_per + m_half + s * m_seg, m_seg), :] = (
                ccw_ref[N_HOPS - 1, pl.ds(s * m_seg, m_seg), :])

        for pair in rdmas.values():
            pair[0].wait_send()
            pair[1].wait_send()

    return pl.pallas_call(
        body,
        out_shape=jax.ShapeDtypeStruct((N_DEV * m_per, n), x.dtype),
        in_specs=[pl.BlockSpec(memory_space=pltpu.VMEM)],
        out_specs=pl.BlockSpec(memory_space=pltpu.VMEM),
        scratch_shapes=[
            pltpu.VMEM((N_HOPS, m_half, n), x.dtype),
            pltpu.VMEM((N_HOPS, m_half, n), x.dtype),
            pltpu.SemaphoreType.DMA((N_HOPS, N_SEG)),
            pltpu.SemaphoreType.DMA((N_HOPS, N_SEG)),
            pltpu.SemaphoreType.DMA((N_HOPS, N_SEG)),
            pltpu.SemaphoreType.DMA((N_HOPS, N_SEG)),
        ],
        compiler_params=pltpu.CompilerParams(collective_id=0),
    )(x)


# device time: 43561 ns/iter; 1.0040x vs baseline; 1.0040x over previous
import jax
import jax.numpy as jnp
from jax import lax
from jax.experimental import pallas as pl
from jax.experimental.pallas import tpu as pltpu

N_DEV = 4
N_HOPS = N_DEV - 1
N_SEG = 2


def kernel(x):
    m_per, n = x.shape
    m_half = m_per // 2
    m_seg = m_half // N_SEG

    def body(x_ref, out_ref, cw_ref, ccw_ref,
             send_cw, recv_cw, send_ccw, recv_ccw):
        my_pos = lax.axis_index("i")
        left = (my_pos - 1) % N_DEV
        right = (my_pos + 1) % N_DEV

        barrier_sem = pltpu.get_barrier_semaphore()
        for nbr in [left, right]:
            pl.semaphore_signal(
                barrier_sem, inc=1,
                device_id=(nbr,), device_id_type=pl.DeviceIdType.MESH,
            )
        pl.semaphore_wait(barrier_sem, 2)

        def make_pair(h, s):
            if h == 0:
                src_cw = x_ref.at[pl.ds(s * m_seg, m_seg), :]
                src_ccw = x_ref.at[pl.ds(m_half + s * m_seg, m_seg), :]
            else:
                src_cw = cw_ref.at[h - 1, pl.ds(s * m_seg, m_seg), :]
                src_ccw = ccw_ref.at[h - 1, pl.ds(s * m_seg, m_seg), :]
            rdma_cw = pltpu.make_async_remote_copy(
                src_ref=src_cw,
                dst_ref=cw_ref.at[h, pl.ds(s * m_seg, m_seg), :],
                send_sem=send_cw.at[h, s],
                recv_sem=recv_cw.at[h, s],
                device_id=(right,),
                device_id_type=pl.DeviceIdType.MESH,
            )
            rdma_ccw = pltpu.make_async_remote_copy(
                src_ref=src_ccw,
                dst_ref=ccw_ref.at[h, pl.ds(s * m_seg, m_seg), :],
                send_sem=send_ccw.at[h, s],
                recv_sem=recv_ccw.at[h, s],
                device_id=(left,),
                device_id_type=pl.DeviceIdType.MESH,
            )
            return rdma_cw, rdma_ccw

        def store_hop(h):
            o_cw = (my_pos - h - 1) % N_DEV
            o_ccw = (my_pos + h + 1) % N_DEV
            out_ref[pl.ds(o_cw * m_per, m_half), :] = cw_ref[h]
            out_ref[pl.ds(o_ccw * m_per + m_half, m_half), :] = ccw_ref[h]

        rdmas = {}
        for s in range(N_SEG):
            pair = make_pair(0, s)
            pair[0].start()
            pair[1].start()
            rdmas[(0, s)] = pair

        out_ref[pl.ds(my_pos * m_per, m_per), :] = x_ref[:, :]

        for h in range(1, N_HOPS):
            for s in range(N_SEG):
                rdmas[(h - 1, s)][0].wait_recv()
                rdmas[(h - 1, s)][1].wait_recv()
                pair = make_pair(h, s)
                pair[0].start()
                pair[1].start()
                rdmas[(h, s)] = pair
            store_hop(h - 1)

        o_cw = (my_pos - N_HOPS) % N_DEV
        o_ccw = (my_pos + N_HOPS) % N_DEV
        for s in range(N_SEG):
            rdmas[(N_HOPS - 1, s)][0].wait_recv()
            rdmas[(N_HOPS - 1, s)][1].wait_recv()
            out_ref[pl.ds(o_cw * m_per + s * m_seg, m_seg), :] = (
                cw_ref[N_HOPS - 1, pl.ds(s * m_seg, m_seg), :])
            out_ref[pl.ds(o_ccw * m_per + m_half + s * m_seg, m_seg), :] = (
                ccw_ref[N_HOPS - 1, pl.ds(s * m_seg, m_seg), :])

        for pair in rdmas.values():
            pair[0].wait_send()
            pair[1].wait_send()

    return pl.pallas_call(
        body,
        out_shape=jax.ShapeDtypeStruct((N_DEV * m_per, n), x.dtype),
        in_specs=[pl.BlockSpec(memory_space=pltpu.VMEM)],
        out_specs=pl.BlockSpec(memory_space=pltpu.VMEM),
        scratch_shapes=[
            pltpu.VMEM((N_HOPS, m_half, n), x.dtype),
            pltpu.VMEM((N_HOPS, m_half, n), x.dtype),
            pltpu.SemaphoreType.DMA((N_HOPS, N_SEG)),
            pltpu.SemaphoreType.DMA((N_HOPS, N_SEG)),
            pltpu.SemaphoreType.DMA((N_HOPS, N_SEG)),
            pltpu.SemaphoreType.DMA((N_HOPS, N_SEG)),
        ],
        compiler_params=pltpu.CompilerParams(collective_id=0),
    )(x)
